# baseline (speedup 1.0000x reference)
import functools

import jax
import jax.numpy as jnp
import numpy as np
from jax import lax
from jax.experimental import pallas as pl
from jax.experimental.pallas import tpu as pltpu
from jax.experimental.pallas import tpu_sc as plsc

_N_NODES = 100000
_E = 6400000
_CUTOFF = 5.0

_NW = 32
_C = 512
_NCHUNK = _E // _C
_TPW = -(-_NCHUNK // _NW)


def _sc_edge_vectors(px, py, pz, sender, receiver, shifts_flat):
    mesh = plsc.VectorSubcoreMesh(core_axis_name="c", subcore_axis_name="s")

    @functools.partial(
        pl.kernel,
        mesh=mesh,
        compiler_params=pltpu.CompilerParams(needs_layout_passes=False),
        out_type=[jax.ShapeDtypeStruct((_E,), jnp.float32) for _ in range(3)],
        scratch_types=[
            pltpu.VMEM((_C,), jnp.int32),
            pltpu.VMEM((_C,), jnp.int32),
            pltpu.VMEM((_C,), jnp.float32),
            pltpu.VMEM((_C,), jnp.float32),
            pltpu.VMEM((_C,), jnp.float32),
            pltpu.VMEM((_C,), jnp.float32),
            pltpu.VMEM((_C,), jnp.float32),
            pltpu.VMEM((_C,), jnp.float32),
            pltpu.VMEM((3 * _C,), jnp.float32),
            pltpu.VMEM((_C,), jnp.float32),
            pltpu.VMEM((_C,), jnp.float32),
            pltpu.VMEM((_C,), jnp.float32),
            pltpu.SemaphoreType.DMA,
        ],
    )
    def k(px_hbm, py_hbm, pz_hbm, send_hbm, recv_hbm, sh_hbm,
          vx_hbm, vy_hbm, vz_hbm,
          sidx, ridx, sxb, syb, szb, rxb, ryb, rzb, shl, ox, oy, oz, sem):
        wid = lax.axis_index("s") * 2 + lax.axis_index("c")
        iota3 = lax.iota(jnp.int32, 16) * 3

        def chunk(t, carry):
            cid = wid + _NW * t

            @pl.when(cid < _NCHUNK)
            def _():
                base = cid * _C
                pltpu.sync_copy(send_hbm.at[pl.ds(base, _C)], sidx)
                pltpu.sync_copy(recv_hbm.at[pl.ds(base, _C)], ridx)
                pltpu.sync_copy(sh_hbm.at[pl.ds(3 * base, 3 * _C)], shl)
                cps = []
                for j in range(_C // 128):
                    sl = pl.ds(j * 128, 128)
                    for tab, idx, dst in ((px_hbm, sidx, sxb),
                                          (py_hbm, sidx, syb),
                                          (pz_hbm, sidx, szb),
                                          (px_hbm, ridx, rxb),
                                          (py_hbm, ridx, ryb),
                                          (pz_hbm, ridx, rzb)):
                        cps.append(pltpu.async_copy(
                            tab.at[idx.at[sl]], dst.at[sl], sem))
                for cp in cps:
                    cp.wait()
                for g in range(_C // 16):
                    s = pl.ds(g * 16, 16)
                    b3 = g * 48
                    hx = plsc.load_gather(shl, [b3 + iota3])
                    hy = plsc.load_gather(shl, [b3 + 1 + iota3])
                    hz = plsc.load_gather(shl, [b3 + 2 + iota3])
                    ox[s] = rxb[s] - sxb[s] + hx
                    oy[s] = ryb[s] - syb[s] + hy
                    oz[s] = rzb[s] - szb[s] + hz
                pltpu.sync_copy(ox, vx_hbm.at[pl.ds(base, _C)])
                pltpu.sync_copy(oy, vy_hbm.at[pl.ds(base, _C)])
                pltpu.sync_copy(oz, vz_hbm.at[pl.ds(base, _C)])
            return carry

        lax.fori_loop(0, _TPW, chunk, 0)

    return k(px, py, pz, sender, receiver, shifts_flat)


_BR = 400


def _tc_radial(vx, vy, vz):
    w = (np.pi * np.arange(1, 7, dtype=np.float32) / _CUTOFF).tolist()
    pref = float(np.sqrt(2.0 / _CUTOFF))
    rows = _E // 128

    def body(vx_ref, vy_ref, vz_ref, len_ref, emb_ref, unit_ref):
        x = vx_ref[...]
        y = vy_ref[...]
        z = vz_ref[...]
        d2 = x * x + y * y + z * z
        l = jnp.sqrt(d2)
        inv = jnp.where(l > 0.0, 1.0 / l, 0.0)
        r = l * (1.0 / _CUTOFF)
        r2 = r * r
        r6 = r2 * r2 * r2
        env = 1.0 + r6 * (-28.0 + 48.0 * r - 21.0 * r2)
        env = jnp.where(l < _CUTOFF, env, 0.0)
        b = (pref * inv) * env
        len_ref[...] = l

        lane = lax.broadcasted_iota(jnp.int32, (_BR, 128), 1)

        def interleave_store(ref, planes):
            n = len(planes)
            for t in range(n):
                f = t * 128 + lane
                idx = f // n
                fm = f % n
                acc = jnp.take_along_axis(planes[0], idx, axis=1)
                for k in range(1, n):
                    g = jnp.take_along_axis(planes[k], idx, axis=1)
                    acc = jnp.where(fm == k, g, acc)
                ref[:, t * 128:(t + 1) * 128] = acc

        interleave_store(unit_ref, [x * inv, y * inv, z * inv])
        theta = w[0] * l
        s1 = jnp.sin(theta)
        c2 = 2.0 * jnp.cos(theta)
        embs = [b * s1]
        sk_m1, sk = s1, c2 * s1
        embs.append(b * sk)
        for _ in range(2, 6):
            sk_m1, sk = sk, c2 * sk - sk_m1
            embs.append(b * sk)
        interleave_store(emb_ref, embs)

    return pl.pallas_call(
        body,
        grid=(rows // _BR,),
        in_specs=[pl.BlockSpec((_BR, 128), lambda i: (i, 0))] * 3,
        out_specs=[
            pl.BlockSpec((_BR, 128), lambda i: (i, 0)),
            pl.BlockSpec((_BR, 768), lambda i: (i, 0)),
            pl.BlockSpec((_BR, 384), lambda i: (i, 0)),
        ],
        out_shape=[
            jax.ShapeDtypeStruct((rows, 128), jnp.float32),
            jax.ShapeDtypeStruct((rows, 768), jnp.float32),
            jax.ShapeDtypeStruct((rows, 384), jnp.float32),
        ],
    )(vx, vy, vz)


def kernel(positions, edge_index, shifts):
    sender = edge_index[0]
    receiver = edge_index[1]
    px = positions[:, 0]
    py = positions[:, 1]
    pz = positions[:, 2]
    vx, vy, vz = _sc_edge_vectors(px, py, pz, sender, receiver,
                                  shifts.reshape(-1))
    rows = _E // 128
    l, emb_t, unit_t = _tc_radial(
        vx.reshape(rows, 128), vy.reshape(rows, 128), vz.reshape(rows, 128))
    lengths = l.reshape(_E, 1)
    emb = emb_t.reshape(_E, 6)
    unit = unit_t.reshape(_E, 3)
    return (lengths, emb, unit)


# speedup vs baseline: 4.7064x; 4.7064x over previous
import functools

import jax
import jax.numpy as jnp
import numpy as np
from jax import lax
from jax.experimental import pallas as pl
from jax.experimental.pallas import tpu as pltpu
from jax.experimental.pallas import tpu_sc as plsc

_N_NODES = 100000
_E = 6400000
_CUTOFF = 5.0

_NW = 32
_C = 512
_NCHUNK = _E // _C
_TPW = -(-_NCHUNK // _NW)


def _sc_edge_vectors(px, py, pz, sender, receiver, shx, shy, shz):
    mesh = plsc.VectorSubcoreMesh(core_axis_name="c", subcore_axis_name="s")

    @functools.partial(
        pl.kernel,
        mesh=mesh,
        compiler_params=pltpu.CompilerParams(needs_layout_passes=False),
        out_type=[jax.ShapeDtypeStruct((_E,), jnp.float32) for _ in range(3)],
        scratch_types=[
            pltpu.VMEM((_C,), jnp.int32),
            pltpu.VMEM((_C,), jnp.int32),
            pltpu.VMEM((_C,), jnp.float32),
            pltpu.VMEM((_C,), jnp.float32),
            pltpu.VMEM((_C,), jnp.float32),
            pltpu.VMEM((_C,), jnp.float32),
            pltpu.VMEM((_C,), jnp.float32),
            pltpu.VMEM((_C,), jnp.float32),
            pltpu.VMEM((_C,), jnp.float32),
            pltpu.VMEM((_C,), jnp.float32),
            pltpu.VMEM((_C,), jnp.float32),
            pltpu.VMEM((_C,), jnp.float32),
            pltpu.VMEM((_C,), jnp.float32),
            pltpu.VMEM((_C,), jnp.float32),
            pltpu.SemaphoreType.DMA,
        ],
    )
    def k(px_hbm, py_hbm, pz_hbm, send_hbm, recv_hbm,
          shx_hbm, shy_hbm, shz_hbm, vx_hbm, vy_hbm, vz_hbm,
          sidx, ridx, sxb, syb, szb, rxb, ryb, rzb, hxb, hyb, hzb,
          ox, oy, oz, sem):
        wid = lax.axis_index("s") * 2 + lax.axis_index("c")

        def chunk(t, carry):
            cid = wid + _NW * t

            @pl.when(cid < _NCHUNK)
            def _():
                base = cid * _C
                pltpu.sync_copy(send_hbm.at[pl.ds(base, _C)], sidx)
                pltpu.sync_copy(recv_hbm.at[pl.ds(base, _C)], ridx)
                pltpu.sync_copy(shx_hbm.at[pl.ds(base, _C)], hxb)
                pltpu.sync_copy(shy_hbm.at[pl.ds(base, _C)], hyb)
                pltpu.sync_copy(shz_hbm.at[pl.ds(base, _C)], hzb)
                cps = []
                for j in range(_C // 128):
                    sl = pl.ds(j * 128, 128)
                    for tab, idx, dst in ((px_hbm, sidx, sxb),
                                          (py_hbm, sidx, syb),
                                          (pz_hbm, sidx, szb),
                                          (px_hbm, ridx, rxb),
                                          (py_hbm, ridx, ryb),
                                          (pz_hbm, ridx, rzb)):
                        cps.append(pltpu.async_copy(
                            tab.at[idx.at[sl]], dst.at[sl], sem))
                for cp in cps:
                    cp.wait()
                for g in range(_C // 16):
                    s = pl.ds(g * 16, 16)
                    ox[s] = rxb[s] - sxb[s] + hxb[s]
                    oy[s] = ryb[s] - syb[s] + hyb[s]
                    oz[s] = rzb[s] - szb[s] + hzb[s]
                pltpu.sync_copy(ox, vx_hbm.at[pl.ds(base, _C)])
                pltpu.sync_copy(oy, vy_hbm.at[pl.ds(base, _C)])
                pltpu.sync_copy(oz, vz_hbm.at[pl.ds(base, _C)])
            return carry

        lax.fori_loop(0, _TPW, chunk, 0)

    return k(px, py, pz, sender, receiver, shx, shy, shz)


_BR = 400


def _tc_radial(vx, vy, vz):
    w = (np.pi * np.arange(1, 7, dtype=np.float32) / _CUTOFF).tolist()
    pref = float(np.sqrt(2.0 / _CUTOFF))
    rows = _E // 128

    def body(vx_ref, vy_ref, vz_ref, len_ref,
             e0_ref, e1_ref, e2_ref, e3_ref, e4_ref, e5_ref,
             ux_ref, uy_ref, uz_ref):
        x = vx_ref[...]
        y = vy_ref[...]
        z = vz_ref[...]
        d2 = x * x + y * y + z * z
        l = jnp.sqrt(d2)
        inv = jnp.where(l > 0.0, 1.0 / l, 0.0)
        r = l * (1.0 / _CUTOFF)
        r2 = r * r
        r6 = r2 * r2 * r2
        env = 1.0 + r6 * (-28.0 + 48.0 * r - 21.0 * r2)
        env = jnp.where(l < _CUTOFF, env, 0.0)
        b = (pref * inv) * env
        len_ref[...] = l
        ux_ref[...] = x * inv
        uy_ref[...] = y * inv
        uz_ref[...] = z * inv
        theta = w[0] * l
        s1 = jnp.sin(theta)
        c2 = 2.0 * jnp.cos(theta)
        e0_ref[...] = b * s1
        sk_m1, sk = s1, c2 * s1
        e1_ref[...] = b * sk
        erefs = [e2_ref, e3_ref, e4_ref, e5_ref]
        for k in range(4):
            sk_m1, sk = sk, c2 * sk - sk_m1
            erefs[k][...] = b * sk

    spec = pl.BlockSpec((_BR, 128), lambda i: (i, 0))
    return pl.pallas_call(
        body,
        grid=(rows // _BR,),
        in_specs=[spec] * 3,
        out_specs=[spec] * 10,
        out_shape=[jax.ShapeDtypeStruct((rows, 128), jnp.float32)] * 10,
    )(vx, vy, vz)


def kernel(positions, edge_index, shifts):
    sender = edge_index[0]
    receiver = edge_index[1]
    px = positions[:, 0]
    py = positions[:, 1]
    pz = positions[:, 2]
    vx, vy, vz = _sc_edge_vectors(px, py, pz, sender, receiver,
                                  shifts[:, 0], shifts[:, 1], shifts[:, 2])
    rows = _E // 128
    outs = _tc_radial(
        vx.reshape(rows, 128), vy.reshape(rows, 128), vz.reshape(rows, 128))
    lengths = outs[0].reshape(_E, 1)
    emb = jnp.stack([outs[1 + k].reshape(_E) for k in range(6)], axis=-1)
    unit = jnp.stack([outs[7 + k].reshape(_E) for k in range(3)], axis=-1)
    return (lengths, emb, unit)


# speedup vs baseline: 8.6248x; 1.8326x over previous
import functools

import jax
import jax.numpy as jnp
import numpy as np
from jax import lax
from jax.experimental import pallas as pl
from jax.experimental.pallas import tpu as pltpu
from jax.experimental.pallas import tpu_sc as plsc

_N_NODES = 100000
_E = 6400000
_CUTOFF = 5.0

_NW = 32
_C = 1024
_NCHUNK = _E // _C
_TPW = -(-_NCHUNK // _NW)


def _sc_edge_vectors(px, py, pz, sender, receiver, shx, shy, shz):
    mesh = plsc.VectorSubcoreMesh(core_axis_name="c", subcore_axis_name="s")

    @functools.partial(
        pl.kernel,
        mesh=mesh,
        compiler_params=pltpu.CompilerParams(needs_layout_passes=False),
        out_type=[jax.ShapeDtypeStruct((_E,), jnp.float32) for _ in range(3)],
        scratch_types=[
            pltpu.VMEM((_C,), jnp.int32),
            pltpu.VMEM((_C,), jnp.int32),
            pltpu.VMEM((_C,), jnp.float32),
            pltpu.VMEM((_C,), jnp.float32),
            pltpu.VMEM((_C,), jnp.float32),
            pltpu.VMEM((_C,), jnp.float32),
            pltpu.VMEM((_C,), jnp.float32),
            pltpu.VMEM((_C,), jnp.float32),
            pltpu.VMEM((_C,), jnp.float32),
            pltpu.VMEM((_C,), jnp.float32),
            pltpu.VMEM((_C,), jnp.float32),
            pltpu.VMEM((_C,), jnp.float32),
            pltpu.VMEM((_C,), jnp.float32),
            pltpu.VMEM((_C,), jnp.float32),
            pltpu.VMEM_SHARED((_N_NODES,), jnp.float32),
            pltpu.VMEM_SHARED((_N_NODES,), jnp.float32),
            pltpu.VMEM_SHARED((_N_NODES,), jnp.float32),
            pltpu.SemaphoreType.DMA,
        ],
    )
    def k(px_hbm, py_hbm, pz_hbm, send_hbm, recv_hbm,
          shx_hbm, shy_hbm, shz_hbm, vx_hbm, vy_hbm, vz_hbm,
          sidx, ridx, sxb, syb, szb, rxb, ryb, rzb, hxb, hyb, hzb,
          ox, oy, oz, pxs, pys, pzs, sem):
        wid = lax.axis_index("s") * 2 + lax.axis_index("c")

        @pl.when(lax.axis_index("s") == 0)
        def _stage():
            pltpu.sync_copy(px_hbm, pxs)
            pltpu.sync_copy(py_hbm, pys)
            pltpu.sync_copy(pz_hbm, pzs)

        plsc.subcore_barrier()

        def chunk(t, carry):
            cid = wid + _NW * t

            @pl.when(cid < _NCHUNK)
            def _():
                base = cid * _C
                pltpu.sync_copy(send_hbm.at[pl.ds(base, _C)], sidx)
                pltpu.sync_copy(recv_hbm.at[pl.ds(base, _C)], ridx)
                pltpu.sync_copy(shx_hbm.at[pl.ds(base, _C)], hxb)
                pltpu.sync_copy(shy_hbm.at[pl.ds(base, _C)], hyb)
                pltpu.sync_copy(shz_hbm.at[pl.ds(base, _C)], hzb)
                cps = []
                for j in range(_C // 128):
                    sl = pl.ds(j * 128, 128)
                    for tab, idx, dst in ((pxs, sidx, sxb),
                                          (pys, sidx, syb),
                                          (pzs, sidx, szb),
                                          (pxs, ridx, rxb),
                                          (pys, ridx, ryb),
                                          (pzs, ridx, rzb)):
                        cps.append(pltpu.async_copy(
                            tab.at[idx.at[sl]], dst.at[sl], sem))
                for cp in cps:
                    cp.wait()
                for g in range(_C // 16):
                    s = pl.ds(g * 16, 16)
                    ox[s] = rxb[s] - sxb[s] + hxb[s]
                    oy[s] = ryb[s] - syb[s] + hyb[s]
                    oz[s] = rzb[s] - szb[s] + hzb[s]
                pltpu.sync_copy(ox, vx_hbm.at[pl.ds(base, _C)])
                pltpu.sync_copy(oy, vy_hbm.at[pl.ds(base, _C)])
                pltpu.sync_copy(oz, vz_hbm.at[pl.ds(base, _C)])
            return carry

        lax.fori_loop(0, _TPW, chunk, 0)

    return k(px, py, pz, sender, receiver, shx, shy, shz)


_BR = 400


def _tc_radial(vx, vy, vz):
    w = (np.pi * np.arange(1, 7, dtype=np.float32) / _CUTOFF).tolist()
    pref = float(np.sqrt(2.0 / _CUTOFF))
    rows = _E // 128

    def body(vx_ref, vy_ref, vz_ref, len_ref,
             e0_ref, e1_ref, e2_ref, e3_ref, e4_ref, e5_ref,
             ux_ref, uy_ref, uz_ref):
        x = vx_ref[...]
        y = vy_ref[...]
        z = vz_ref[...]
        d2 = x * x + y * y + z * z
        l = jnp.sqrt(d2)
        inv = jnp.where(l > 0.0, 1.0 / l, 0.0)
        r = l * (1.0 / _CUTOFF)
        r2 = r * r
        r6 = r2 * r2 * r2
        env = 1.0 + r6 * (-28.0 + 48.0 * r - 21.0 * r2)
        env = jnp.where(l < _CUTOFF, env, 0.0)
        b = (pref * inv) * env
        len_ref[...] = l
        ux_ref[...] = x * inv
        uy_ref[...] = y * inv
        uz_ref[...] = z * inv
        theta = w[0] * l
        s1 = jnp.sin(theta)
        c2 = 2.0 * jnp.cos(theta)
        e0_ref[...] = b * s1
        sk_m1, sk = s1, c2 * s1
        e1_ref[...] = b * sk
        erefs = [e2_ref, e3_ref, e4_ref, e5_ref]
        for k in range(4):
            sk_m1, sk = sk, c2 * sk - sk_m1
            erefs[k][...] = b * sk

    spec = pl.BlockSpec((_BR, 128), lambda i: (i, 0))
    return pl.pallas_call(
        body,
        grid=(rows // _BR,),
        in_specs=[spec] * 3,
        out_specs=[spec] * 10,
        out_shape=[jax.ShapeDtypeStruct((rows, 128), jnp.float32)] * 10,
    )(vx, vy, vz)


def kernel(positions, edge_index, shifts):
    sender = edge_index[0]
    receiver = edge_index[1]
    px = positions[:, 0]
    py = positions[:, 1]
    pz = positions[:, 2]
    vx, vy, vz = _sc_edge_vectors(px, py, pz, sender, receiver,
                                  shifts[:, 0], shifts[:, 1], shifts[:, 2])
    rows = _E // 128
    outs = _tc_radial(
        vx.reshape(rows, 128), vy.reshape(rows, 128), vz.reshape(rows, 128))
    lengths = outs[0].reshape(_E, 1)
    emb = jnp.stack([outs[1 + k].reshape(_E) for k in range(6)], axis=-1)
    unit = jnp.stack([outs[7 + k].reshape(_E) for k in range(3)], axis=-1)
    return (lengths, emb, unit)


# speedup vs baseline: 10.7362x; 1.2448x over previous
import functools

import jax
import jax.numpy as jnp
import numpy as np
from jax import lax
from jax.experimental import pallas as pl
from jax.experimental.pallas import tpu as pltpu
from jax.experimental.pallas import tpu_sc as plsc

_N_NODES = 100000
_E = 6400000
_CUTOFF = 5.0

_NW = 32
_C = 1024
_IDXW = 512
_NCHUNK = _E // _C
_TPW = -(-_NCHUNK // _NW)


def _sc_edge_vectors(px, py, pz, sender, receiver, shx, shy, shz):
    mesh = plsc.VectorSubcoreMesh(core_axis_name="c", subcore_axis_name="s")

    @functools.partial(
        pl.kernel,
        mesh=mesh,
        compiler_params=pltpu.CompilerParams(needs_layout_passes=False),
        out_type=[jax.ShapeDtypeStruct((_E,), jnp.float32) for _ in range(3)],
        scratch_types=[
            pltpu.VMEM((_C,), jnp.int32),
            pltpu.VMEM((_C,), jnp.int32),
            pltpu.VMEM((_C,), jnp.float32),
            pltpu.VMEM((_C,), jnp.float32),
            pltpu.VMEM((_C,), jnp.float32),
            pltpu.VMEM((_C,), jnp.float32),
            pltpu.VMEM((_C,), jnp.float32),
            pltpu.VMEM((_C,), jnp.float32),
            pltpu.VMEM((_C,), jnp.float32),
            pltpu.VMEM((_C,), jnp.float32),
            pltpu.VMEM((_C,), jnp.float32),
            pltpu.VMEM((_C,), jnp.float32),
            pltpu.VMEM((_C,), jnp.float32),
            pltpu.VMEM((_C,), jnp.float32),
            pltpu.VMEM_SHARED((_N_NODES,), jnp.float32),
            pltpu.VMEM_SHARED((_N_NODES,), jnp.float32),
            pltpu.VMEM_SHARED((_N_NODES,), jnp.float32),
            pltpu.SemaphoreType.DMA,
        ],
    )
    def k(px_hbm, py_hbm, pz_hbm, send_hbm, recv_hbm,
          shx_hbm, shy_hbm, shz_hbm, vx_hbm, vy_hbm, vz_hbm,
          sidx, ridx, sxb, syb, szb, rxb, ryb, rzb, hxb, hyb, hzb,
          ox, oy, oz, pxs, pys, pzs, sem):
        wid = lax.axis_index("s") * 2 + lax.axis_index("c")

        @pl.when(lax.axis_index("s") == 0)
        def _stage():
            pltpu.sync_copy(px_hbm, pxs)
            pltpu.sync_copy(py_hbm, pys)
            pltpu.sync_copy(pz_hbm, pzs)

        plsc.subcore_barrier()

        def chunk(t, carry):
            cid = wid + _NW * t

            @pl.when(cid < _NCHUNK)
            def _():
                base = cid * _C
                ins = [
                    pltpu.async_copy(send_hbm.at[pl.ds(base, _C)], sidx, sem),
                    pltpu.async_copy(recv_hbm.at[pl.ds(base, _C)], ridx, sem),
                    pltpu.async_copy(shx_hbm.at[pl.ds(base, _C)], hxb, sem),
                    pltpu.async_copy(shy_hbm.at[pl.ds(base, _C)], hyb, sem),
                    pltpu.async_copy(shz_hbm.at[pl.ds(base, _C)], hzb, sem),
                ]
                for cp in ins:
                    cp.wait()
                cps = []
                for j in range(_C // _IDXW):
                    sl = pl.ds(j * _IDXW, _IDXW)
                    for tab, idx, dst in ((pxs, sidx, sxb),
                                          (pys, sidx, syb),
                                          (pzs, sidx, szb),
                                          (pxs, ridx, rxb),
                                          (pys, ridx, ryb),
                                          (pzs, ridx, rzb)):
                        cps.append(pltpu.async_copy(
                            tab.at[idx.at[sl]], dst.at[sl], sem))
                for cp in cps:
                    cp.wait()
                for g in range(_C // 16):
                    s = pl.ds(g * 16, 16)
                    ox[s] = rxb[s] - sxb[s] + hxb[s]
                    oy[s] = ryb[s] - syb[s] + hyb[s]
                    oz[s] = rzb[s] - szb[s] + hzb[s]
                outs = [
                    pltpu.async_copy(ox, vx_hbm.at[pl.ds(base, _C)], sem),
                    pltpu.async_copy(oy, vy_hbm.at[pl.ds(base, _C)], sem),
                    pltpu.async_copy(oz, vz_hbm.at[pl.ds(base, _C)], sem),
                ]
                for cp in outs:
                    cp.wait()
            return carry

        lax.fori_loop(0, _TPW, chunk, 0)

    return k(px, py, pz, sender, receiver, shx, shy, shz)


_BR = 400


def _tc_radial(vx, vy, vz):
    w = (np.pi * np.arange(1, 7, dtype=np.float32) / _CUTOFF).tolist()
    pref = float(np.sqrt(2.0 / _CUTOFF))
    rows = _E // 128

    def body(vx_ref, vy_ref, vz_ref, len_ref,
             e0_ref, e1_ref, e2_ref, e3_ref, e4_ref, e5_ref,
             ux_ref, uy_ref, uz_ref):
        x = vx_ref[...]
        y = vy_ref[...]
        z = vz_ref[...]
        d2 = x * x + y * y + z * z
        l = jnp.sqrt(d2)
        inv = jnp.where(l > 0.0, 1.0 / l, 0.0)
        r = l * (1.0 / _CUTOFF)
        r2 = r * r
        r6 = r2 * r2 * r2
        env = 1.0 + r6 * (-28.0 + 48.0 * r - 21.0 * r2)
        env = jnp.where(l < _CUTOFF, env, 0.0)
        b = (pref * inv) * env
        len_ref[...] = l
        ux_ref[...] = x * inv
        uy_ref[...] = y * inv
        uz_ref[...] = z * inv
        theta = w[0] * l
        s1 = jnp.sin(theta)
        c2 = 2.0 * jnp.cos(theta)
        e0_ref[...] = b * s1
        sk_m1, sk = s1, c2 * s1
        e1_ref[...] = b * sk
        erefs = [e2_ref, e3_ref, e4_ref, e5_ref]
        for k in range(4):
            sk_m1, sk = sk, c2 * sk - sk_m1
            erefs[k][...] = b * sk

    spec = pl.BlockSpec((_BR, 128), lambda i: (i, 0))
    return pl.pallas_call(
        body,
        grid=(rows // _BR,),
        in_specs=[spec] * 3,
        out_specs=[spec] * 10,
        out_shape=[jax.ShapeDtypeStruct((rows, 128), jnp.float32)] * 10,
    )(vx, vy, vz)


def kernel(positions, edge_index, shifts):
    sender = edge_index[0]
    receiver = edge_index[1]
    px = positions[:, 0]
    py = positions[:, 1]
    pz = positions[:, 2]
    vx, vy, vz = _sc_edge_vectors(px, py, pz, sender, receiver,
                                  shifts[:, 0], shifts[:, 1], shifts[:, 2])
    rows = _E // 128
    outs = _tc_radial(
        vx.reshape(rows, 128), vy.reshape(rows, 128), vz.reshape(rows, 128))
    lengths = outs[0].reshape(_E, 1)
    emb = jnp.stack([outs[1 + k].reshape(_E) for k in range(6)], axis=-1)
    unit = jnp.stack([outs[7 + k].reshape(_E) for k in range(3)], axis=-1)
    return (lengths, emb, unit)
